# baseline (speedup 1.0000x reference)
import jax
import jax.numpy as jnp
from jax import lax
from jax.experimental import pallas as pl
from jax.experimental.pallas import tpu as pltpu
from jax.experimental.pallas import tpu_sc as plsc

EMBED_DIM = 16
BLOCK_B = 128
ROWS_PER_128 = 128 // EMBED_DIM


def kernel(follow_inputs, table):
    batch, hist = follow_inputs.shape
    vocab, d = table.shape
    idx_t = jnp.transpose(follow_inputs)
    tbl = jnp.reshape(table, (vocab * d // 128, 128))

    nb = batch // BLOCK_B
    mesh = plsc.VectorSubcoreMesh(core_axis_name="c", subcore_axis_name="s")

    @pl.kernel(
        out_type=jax.ShapeDtypeStruct((hist, EMBED_DIM, batch), table.dtype),
        mesh=mesh,
        scratch_types=[
            pltpu.VMEM((BLOCK_B,), jnp.int32),
            pltpu.VMEM((BLOCK_B,), jnp.int32),
            pltpu.VMEM((BLOCK_B, 128), jnp.float32),
            pltpu.SemaphoreType.DMA,
        ],
        compiler_params=pltpu.CompilerParams(needs_layout_passes=False),
    )
    def gather_kernel(tbl_hbm, idx_hbm, out_hbm, coarse_v, off_v, rows_v, sem):
        def body(idx_vmem, out_vmem):
            @pl.loop(0, BLOCK_B // 16)
            def _(m):
                v = idx_vmem[0, pl.ds(m * 16, 16)]
                coarse_v[pl.ds(m * 16, 16)] = v >> 3
                off_v[pl.ds(m * 16, 16)] = (v & 7) * 16

            pltpu.async_copy(tbl_hbm.at[coarse_v], rows_v, sem).wait()

            @pl.loop(0, EMBED_DIM)
            def _(c):
                @pl.loop(0, BLOCK_B // 16)
                def _(m):
                    col = plsc.load_gather(
                        rows_v,
                        [
                            m * 16 + lax.iota(jnp.int32, 16),
                            off_v[pl.ds(m * 16, 16)] + c,
                        ],
                    )
                    out_vmem[0, c, pl.ds(m * 16, 16)] = col

        pltpu.emit_pipeline(
            body,
            grid=(hist * nb,),
            in_specs=[
                pl.BlockSpec((1, BLOCK_B), index_map=lambda i: (i // nb, i % nb))
            ],
            out_specs=[
                pl.BlockSpec(
                    (1, EMBED_DIM, BLOCK_B),
                    index_map=lambda i: (i // nb, 0, i % nb),
                )
            ],
            core_axis_name=("c", "s"),
            dimension_semantics=(pltpu.PARALLEL,),
        )(idx_hbm, out_hbm)

    out = gather_kernel(tbl, idx_t)
    return jnp.transpose(out, (2, 0, 1))


# speedup vs baseline: 1.4976x; 1.4976x over previous
import jax
import jax.numpy as jnp
from jax import lax
from jax.experimental import pallas as pl
from jax.experimental.pallas import tpu as pltpu
from jax.experimental.pallas import tpu_sc as plsc

EMBED_DIM = 16
BLOCK_B = 256
ROWS_PER_128 = 128 // EMBED_DIM


def kernel(follow_inputs, table):
    batch, hist = follow_inputs.shape
    vocab, d = table.shape
    idx_t = jnp.transpose(follow_inputs)
    tbl = jnp.reshape(table, (vocab * d // 128, 128))

    nb = batch // BLOCK_B
    mesh = plsc.VectorSubcoreMesh(core_axis_name="c", subcore_axis_name="s")

    @pl.kernel(
        out_type=jax.ShapeDtypeStruct((hist, EMBED_DIM, batch), table.dtype),
        mesh=mesh,
        scratch_types=[
            pltpu.VMEM((BLOCK_B,), jnp.int32),
            pltpu.VMEM((BLOCK_B,), jnp.int32),
            pltpu.VMEM((BLOCK_B, 128), jnp.float32),
            pltpu.SemaphoreType.DMA,
        ],
        compiler_params=pltpu.CompilerParams(needs_layout_passes=False),
    )
    def gather_kernel(tbl_hbm, idx_hbm, out_hbm, coarse_v, off_v, rows_v, sem):
        def body(idx_vmem, out_vmem):
            @pl.loop(0, BLOCK_B // 16)
            def _(m):
                v = idx_vmem[0, pl.ds(m * 16, 16)]
                coarse_v[pl.ds(m * 16, 16)] = v >> 3
                off_v[pl.ds(m * 16, 16)] = (v & 7) * 16

            pltpu.async_copy(tbl_hbm.at[coarse_v], rows_v, sem).wait()

            @plsc.parallel_loop(0, BLOCK_B // 16, unroll=2)
            def _(m):
                rows16 = m * 16 + lax.iota(jnp.int32, 16)
                off16 = off_v[pl.ds(m * 16, 16)]

                @plsc.parallel_loop(0, EMBED_DIM, unroll=4)
                def _(c):
                    col = plsc.load_gather(rows_v, [rows16, off16 + c])
                    out_vmem[0, c, pl.ds(m * 16, 16)] = col

        pltpu.emit_pipeline(
            body,
            grid=(hist * nb,),
            in_specs=[
                pl.BlockSpec((1, BLOCK_B), index_map=lambda i: (i // nb, i % nb))
            ],
            out_specs=[
                pl.BlockSpec(
                    (1, EMBED_DIM, BLOCK_B),
                    index_map=lambda i: (i // nb, 0, i % nb),
                )
            ],
            core_axis_name=("c", "s"),
            dimension_semantics=(pltpu.PARALLEL,),
        )(idx_hbm, out_hbm)

    out = gather_kernel(tbl, idx_t)
    return jnp.transpose(out, (2, 0, 1))


# speedup vs baseline: 1.5719x; 1.0497x over previous
import jax
import jax.numpy as jnp
from jax import lax
from jax.experimental import pallas as pl
from jax.experimental.pallas import tpu as pltpu
from jax.experimental.pallas import tpu_sc as plsc

EMBED_DIM = 16
BLOCK_B = 512
ROWS_PER_128 = 128 // EMBED_DIM


def kernel(follow_inputs, table):
    batch, hist = follow_inputs.shape
    vocab, d = table.shape
    idx_t = jnp.transpose(follow_inputs)
    tbl = jnp.reshape(table, (vocab * d // 128, 128))

    nb = batch // BLOCK_B
    mesh = plsc.VectorSubcoreMesh(core_axis_name="c", subcore_axis_name="s")

    @pl.kernel(
        out_type=jax.ShapeDtypeStruct((hist, EMBED_DIM, batch), table.dtype),
        mesh=mesh,
        scratch_types=[
            pltpu.VMEM((BLOCK_B,), jnp.int32),
            pltpu.VMEM((BLOCK_B,), jnp.int32),
            pltpu.VMEM((BLOCK_B, 128), jnp.float32),
            pltpu.SemaphoreType.DMA,
        ],
        compiler_params=pltpu.CompilerParams(needs_layout_passes=False),
    )
    def gather_kernel(tbl_hbm, idx_hbm, out_hbm, coarse_v, off_v, rows_v, sem):
        def body(idx_vmem, out_vmem):
            @pl.loop(0, BLOCK_B // 16)
            def _(m):
                v = idx_vmem[0, pl.ds(m * 16, 16)]
                coarse_v[pl.ds(m * 16, 16)] = v >> 3
                off_v[pl.ds(m * 16, 16)] = (v & 7) * 16

            pltpu.async_copy(tbl_hbm.at[coarse_v], rows_v, sem).wait()

            @plsc.parallel_loop(0, BLOCK_B // 16, unroll=2)
            def _(m):
                rows16 = m * 16 + lax.iota(jnp.int32, 16)
                off16 = off_v[pl.ds(m * 16, 16)]

                @plsc.parallel_loop(0, EMBED_DIM, unroll=4)
                def _(c):
                    col = plsc.load_gather(rows_v, [rows16, off16 + c])
                    out_vmem[0, c, pl.ds(m * 16, 16)] = col

        pltpu.emit_pipeline(
            body,
            grid=(hist * nb,),
            in_specs=[
                pl.BlockSpec((1, BLOCK_B), index_map=lambda i: (i // nb, i % nb))
            ],
            out_specs=[
                pl.BlockSpec(
                    (1, EMBED_DIM, BLOCK_B),
                    index_map=lambda i: (i // nb, 0, i % nb),
                )
            ],
            core_axis_name=("c", "s"),
            dimension_semantics=(pltpu.PARALLEL,),
        )(idx_hbm, out_hbm)

    out = gather_kernel(tbl, idx_t)
    return jnp.transpose(out, (2, 0, 1))
